# baseline (speedup 1.0000x reference)
import functools

import jax
import jax.numpy as jnp
from jax import lax
from jax.experimental import pallas as pl
from jax.experimental.pallas import tpu as pltpu
from jax.experimental.pallas import tpu_sc as plsc

Q = 1024
D = 32
K_TOTAL = 100000
KT = 1024
K_PAD = 100352
NSTEP = K_PAD // KT
CB = 16
NB = K_PAD // CB
CA = 256
NA = K_PAD // CA
TOPK = 32
NCAND = TOPK * CB
BIG = 3.0e38
IBIG = 2**31 - 1


def _s0_body(q_ref, out_ref):
    q = q_ref[...]
    out_ref[...] = jnp.sum(q * q, axis=1)[None, :]


def _s0(queries):
    return pl.pallas_call(
        _s0_body,
        out_shape=jax.ShapeDtypeStruct((1, Q), jnp.float32),
    )(queries)


def _s1_body(qt_ref, qsq_ref, k_ref, mb_ref, ma_ref, ksq_ref):
    step = pl.program_id(0)
    qt = qt_ref[...]
    qsq = qsq_ref[...]
    k = k_ref[...]
    k_sq = jnp.sum(k * k, axis=1, keepdims=True)
    dots = lax.dot_general(k, qt, (((1,), (0,)), ((), ())),
                           preferred_element_type=jnp.float32)
    dist = (qsq - 2.0 * dots) + k_sq
    row = step * KT + lax.broadcasted_iota(jnp.int32, (KT, 1), 0)
    dist = jnp.where(row < K_TOTAL, dist, BIG)
    bmin = jnp.min(dist.reshape(KT // CB, CB, Q), axis=1)
    mb_ref[0] = bmin
    ma_ref[0] = jnp.min(bmin.reshape(KT // CA, CA // CB, Q), axis=1)
    ksq_ref[0] = k_sq


def _s1(queries_t, q_sq, keys_p):
    return pl.pallas_call(
        _s1_body,
        grid=(NSTEP,),
        in_specs=[
            pl.BlockSpec((D, Q), lambda i: (0, 0)),
            pl.BlockSpec((1, Q), lambda i: (0, 0)),
            pl.BlockSpec((KT, D), lambda i: (i, 0)),
        ],
        out_specs=[
            pl.BlockSpec((1, KT // CB, Q), lambda i: (i, 0, 0)),
            pl.BlockSpec((1, KT // CA, Q), lambda i: (i, 0, 0)),
            pl.BlockSpec((1, KT, 1), lambda i: (i, 0, 0)),
        ],
        out_shape=[
            jax.ShapeDtypeStruct((NSTEP, KT // CB, Q), jnp.float32),
            jax.ShapeDtypeStruct((NSTEP, KT // CA, Q), jnp.float32),
            jax.ShapeDtypeStruct((NSTEP, KT, 1), jnp.float32),
        ],
    )(queries_t, q_sq, keys_p)


def _extract_body(v_ref, g_ref, outv_ref, outi_ref):
    V = v_ref[...]
    G = g_ref[...]
    for i in range(TOPK):
        m = jnp.min(V, axis=0)
        hit = V == m[None, :]
        gi = jnp.min(jnp.where(hit, G, IBIG), axis=0)
        outv_ref[i] = m
        outi_ref[i] = gi
        V = jnp.where(hit & (G == gi[None, :]), BIG, V)


def _extract(vals_t, ids_t):
    return pl.pallas_call(
        _extract_body,
        out_shape=[
            jax.ShapeDtypeStruct((TOPK, Q), jnp.float32),
            jax.ShapeDtypeStruct((TOPK, Q), jnp.int32),
        ],
    )(vals_t, ids_t)


def _sc_gather(table, idx, width, chunk):
    b = idx.shape[0]
    info = plsc.get_sparse_core_info()
    nw = info.num_cores * info.num_subcores
    n = b // nw
    mesh = plsc.VectorSubcoreMesh(core_axis_name="c", subcore_axis_name="s")

    @functools.partial(
        pl.kernel, mesh=mesh,
        compiler_params=pltpu.CompilerParams(use_tc_tiling_on_sc=False),
        out_type=jax.ShapeDtypeStruct((b, width), jnp.float32),
        scratch_types=[
            pltpu.VMEM((chunk,), jnp.int32),
            pltpu.VMEM((chunk, width), jnp.float32),
            pltpu.SemaphoreType.DMA,
        ],
    )
    def k(idx_hbm, table_hbm, out_hbm, idx_v, rows_v, sem):
        wid = lax.axis_index("s") * info.num_cores + lax.axis_index("c")
        base = wid * n
        for j in range(n // chunk):
            off = base + j * chunk
            pltpu.sync_copy(idx_hbm.at[pl.ds(off, chunk)], idx_v)
            pltpu.async_copy(table_hbm.at[idx_v], rows_v, sem).wait()
            pltpu.sync_copy(rows_v, out_hbm.at[pl.ds(off, chunk)])

    return k(idx, table)


QB = 8
CBLK = QB * NCAND


def _s6a_body(c_ref, q_ref, ksq_ref, out_ref):
    cand = c_ref[...]
    qb = q_ref[...]
    ksq = ksq_ref[...]
    q_sq = jnp.sum(qb * qb, axis=1, keepdims=True)
    cand_t = jnp.swapaxes(cand, 0, 1)
    dots = lax.dot_general(qb, cand_t, (((1,), (0,)), ((), ())),
                           preferred_element_type=jnp.float32)
    t3 = dots.reshape(QB, QB, NCAND)
    ii = lax.broadcasted_iota(jnp.int32, (QB, QB, NCAND), 0)
    jj = lax.broadcasted_iota(jnp.int32, (QB, QB, NCAND), 1)
    own = jnp.sum(jnp.where(ii == jj, t3, 0.0), axis=1)
    out_ref[...] = (q_sq - 2.0 * own) + ksq


def _s6a(cand, queries, ksq_cand):
    return pl.pallas_call(
        _s6a_body,
        grid=(Q // QB,),
        in_specs=[
            pl.BlockSpec((CBLK, D), lambda i: (i, 0)),
            pl.BlockSpec((QB, D), lambda i: (i, 0)),
            pl.BlockSpec((QB, NCAND), lambda i: (i, 0)),
        ],
        out_specs=pl.BlockSpec((QB, NCAND), lambda i: (i, 0)),
        out_shape=jax.ShapeDtypeStruct((Q, NCAND), jnp.float32),
    )(cand, queries, ksq_cand)


def kernel(queries, keys, k):
    keys_p = jnp.pad(keys, ((0, K_PAD - K_TOTAL), (0, 0)))
    queries_t = queries.T

    q_sq = _s0(queries)

    m_b3, m_a3, ksq3 = _s1(queries_t, q_sq, keys_p)
    m_at = m_a3.reshape(NA, Q)

    a_ids = lax.broadcasted_iota(jnp.int32, (NA, Q), 0)
    _, sel_a_t = _extract(m_at, a_ids)
    sel_a = sel_a_t.T

    qcol = jnp.arange(Q, dtype=jnp.int32)[:, None]
    gb = (sel_a[:, :, None] * CB +
          jnp.arange(CB, dtype=jnp.int32)[None, None, :]).reshape(Q, NCAND)
    bm_idx = (gb * Q + qcol).reshape(-1)
    bm = _sc_gather(m_b3.reshape(NSTEP * 64 * Q, 1), bm_idx, 1, 2048)
    bm_t = bm.reshape(Q, NCAND).T

    _, sel_b_t = _extract(bm_t, gb.T)
    sel_b = sel_b_t.T

    ck = (sel_b[:, :, None] * CB +
          jnp.arange(CB, dtype=jnp.int32)[None, None, :]).reshape(Q, NCAND)
    ck_flat = ck.reshape(-1)
    cand = _sc_gather(keys_p, ck_flat, D, 2048)
    ksq_cand = _sc_gather(ksq3.reshape(K_PAD, 1), ck_flat, 1, 2048)

    dist_c = _s6a(cand, queries, ksq_cand.reshape(Q, NCAND))
    vals_t, idx_t = _extract(dist_c.T, ck.T)
    top_vals = vals_t.T
    top_idx = idx_t.T + (k * 0)

    gathered = _sc_gather(keys_p, top_idx.reshape(-1), D, 1024)
    return gathered.reshape(Q, TOPK, D), top_vals, top_idx


# speedup vs baseline: 10.4938x; 10.4938x over previous
import functools

import jax
import jax.numpy as jnp
from jax import lax
from jax.experimental import pallas as pl
from jax.experimental.pallas import tpu as pltpu
from jax.experimental.pallas import tpu_sc as plsc

Q = 1024
D = 32
K_TOTAL = 100000
KT = 1024
K_PAD = 100352
NSTEP = K_PAD // KT
CB = 16
NB = K_PAD // CB
CA = 256
NA = K_PAD // CA
TOPK = 32
NCAND = TOPK * CB
BIG = 3.0e38
IBIG = 2**31 - 1


def _s0_body(q_ref, out_ref):
    q = q_ref[...]
    out_ref[...] = jnp.sum(q * q, axis=1)[None, :]


def _s0(queries):
    return pl.pallas_call(
        _s0_body,
        out_shape=jax.ShapeDtypeStruct((1, Q), jnp.float32),
    )(queries)


def _s1_body(qt_ref, qsq_ref, k_ref, mb_ref, ma_ref, ksq_ref):
    step = pl.program_id(0)
    qt = qt_ref[...]
    qsq = qsq_ref[...]
    k = k_ref[...]
    k_sq = jnp.sum(k * k, axis=1, keepdims=True)
    dots = lax.dot_general(k, qt, (((1,), (0,)), ((), ())),
                           preferred_element_type=jnp.float32)
    dist = (qsq - 2.0 * dots) + k_sq
    row = step * KT + lax.broadcasted_iota(jnp.int32, (KT, 1), 0)
    dist = jnp.where(row < K_TOTAL, dist, BIG)
    bmin = jnp.min(dist.reshape(KT // CB, CB, Q), axis=1)
    mb_ref[0] = jnp.swapaxes(bmin, 0, 1)
    ma_ref[0] = jnp.min(bmin.reshape(KT // CA, CA // CB, Q), axis=1)
    ksq_ref[0] = k_sq


def _s1(queries_t, q_sq, keys_p):
    return pl.pallas_call(
        _s1_body,
        grid=(NSTEP,),
        in_specs=[
            pl.BlockSpec((D, Q), lambda i: (0, 0)),
            pl.BlockSpec((1, Q), lambda i: (0, 0)),
            pl.BlockSpec((KT, D), lambda i: (i, 0)),
        ],
        out_specs=[
            pl.BlockSpec((1, Q, KT // CB), lambda i: (i, 0, 0)),
            pl.BlockSpec((1, KT // CA, Q), lambda i: (i, 0, 0)),
            pl.BlockSpec((1, KT, 1), lambda i: (i, 0, 0)),
        ],
        out_shape=[
            jax.ShapeDtypeStruct((NSTEP, Q, KT // CB), jnp.float32),
            jax.ShapeDtypeStruct((NSTEP, KT // CA, Q), jnp.float32),
            jax.ShapeDtypeStruct((NSTEP, KT, 1), jnp.float32),
        ],
    )(queries_t, q_sq, keys_p)


def _extract_body(v_ref, g_ref, outv_ref, outi_ref):
    V = v_ref[...]
    G = g_ref[...]
    for i in range(TOPK):
        m = jnp.min(V, axis=0)
        hit = V == m[None, :]
        gi = jnp.min(jnp.where(hit, G, IBIG), axis=0)
        outv_ref[i] = m
        outi_ref[i] = gi
        V = jnp.where(hit & (G == gi[None, :]), BIG, V)


def _extract(vals_t, ids_t):
    return pl.pallas_call(
        _extract_body,
        out_shape=[
            jax.ShapeDtypeStruct((TOPK, Q), jnp.float32),
            jax.ShapeDtypeStruct((TOPK, Q), jnp.int32),
        ],
    )(vals_t, ids_t)


def _sc_gather(table, idx, width, chunk):
    b = idx.shape[0]
    info = plsc.get_sparse_core_info()
    nw = info.num_cores * info.num_subcores
    n = b // nw
    mesh = plsc.VectorSubcoreMesh(core_axis_name="c", subcore_axis_name="s")

    @functools.partial(
        pl.kernel, mesh=mesh,
        compiler_params=pltpu.CompilerParams(use_tc_tiling_on_sc=False),
        out_type=jax.ShapeDtypeStruct((b, width), jnp.float32),
        scratch_types=[
            pltpu.VMEM((chunk,), jnp.int32),
            pltpu.VMEM((chunk, width), jnp.float32),
            pltpu.SemaphoreType.DMA,
        ],
    )
    def k(idx_hbm, table_hbm, out_hbm, idx_v, rows_v, sem):
        wid = lax.axis_index("s") * info.num_cores + lax.axis_index("c")
        base = wid * n
        for j in range(n // chunk):
            off = base + j * chunk
            pltpu.sync_copy(idx_hbm.at[pl.ds(off, chunk)], idx_v)
            pltpu.async_copy(table_hbm.at[idx_v], rows_v, sem).wait()
            pltpu.sync_copy(rows_v, out_hbm.at[pl.ds(off, chunk)])

    return k(idx, table)


QB = 8
CBLK = QB * NCAND


def _s6a_body(c_ref, q_ref, ksq_ref, out_ref):
    cand = c_ref[...]
    qb = q_ref[...]
    ksq = ksq_ref[...]
    q_sq = jnp.sum(qb * qb, axis=1, keepdims=True)
    cand_t = jnp.swapaxes(cand, 0, 1)
    dots = lax.dot_general(qb, cand_t, (((1,), (0,)), ((), ())),
                           preferred_element_type=jnp.float32)
    t3 = dots.reshape(QB, QB, NCAND)
    ii = lax.broadcasted_iota(jnp.int32, (QB, QB, NCAND), 0)
    jj = lax.broadcasted_iota(jnp.int32, (QB, QB, NCAND), 1)
    own = jnp.sum(jnp.where(ii == jj, t3, 0.0), axis=1)
    out_ref[...] = (q_sq - 2.0 * own) + ksq


def _s6a(cand, queries, ksq_cand):
    return pl.pallas_call(
        _s6a_body,
        grid=(Q // QB,),
        in_specs=[
            pl.BlockSpec((CBLK, D), lambda i: (i, 0)),
            pl.BlockSpec((QB, D), lambda i: (i, 0)),
            pl.BlockSpec((QB, NCAND), lambda i: (i, 0)),
        ],
        out_specs=pl.BlockSpec((QB, NCAND), lambda i: (i, 0)),
        out_shape=jax.ShapeDtypeStruct((Q, NCAND), jnp.float32),
    )(cand, queries, ksq_cand)


def kernel(queries, keys, k):
    keys_p = jnp.pad(keys, ((0, K_PAD - K_TOTAL), (0, 0)))
    queries_t = queries.T

    q_sq = _s0(queries)

    m_b3, m_a3, ksq3 = _s1(queries_t, q_sq, keys_p)
    m_at = m_a3.reshape(NA, Q)

    a_ids = lax.broadcasted_iota(jnp.int32, (NA, Q), 0)
    _, sel_a_t = _extract(m_at, a_ids)
    sel_a = sel_a_t.T

    qcol = jnp.arange(Q, dtype=jnp.int32)[:, None]
    gb = (sel_a[:, :, None] * CB +
          jnp.arange(CB, dtype=jnp.int32)[None, None, :]).reshape(Q, NCAND)
    bm_idx = ((sel_a // 4) * 4096 + qcol * 4 + sel_a % 4).reshape(-1)
    bm = _sc_gather(m_b3.reshape(NSTEP * Q * 4, CB), bm_idx, CB, 1024)
    bm_t = bm.reshape(Q, NCAND).T

    _, sel_b_t = _extract(bm_t, gb.T)
    sel_b = sel_b_t.T

    ck = (sel_b[:, :, None] * CB +
          jnp.arange(CB, dtype=jnp.int32)[None, None, :]).reshape(Q, NCAND)
    ck_flat = ck.reshape(-1)
    cand = _sc_gather(keys_p, ck_flat, D, 2048)
    ksq_cand = _sc_gather(ksq3.reshape(NB, CB), sel_b.reshape(-1), CB, 1024)

    dist_c = _s6a(cand, queries, ksq_cand.reshape(Q, NCAND))
    vals_t, idx_t = _extract(dist_c.T, ck.T)
    top_vals = vals_t.T
    top_idx = idx_t.T + (k * 0)

    gathered = _sc_gather(keys_p, top_idx.reshape(-1), D, 1024)
    return gathered.reshape(Q, TOPK, D), top_vals, top_idx


# speedup vs baseline: 11.2267x; 1.0698x over previous
import functools

import jax
import jax.numpy as jnp
from jax import lax
from jax.experimental import pallas as pl
from jax.experimental.pallas import tpu as pltpu
from jax.experimental.pallas import tpu_sc as plsc

Q = 1024
D = 32
K_TOTAL = 100000
KT = 2048
K_PAD = 100352
NSTEP = K_PAD // KT
APS = KT // 256
CB = 16
NB = K_PAD // CB
CA = 256
NA = K_PAD // CA
TOPK = 32
NCAND = TOPK * CB
BIG = 3.0e38
IBIG = 2**31 - 1


def _s0_body(q_ref, out_ref):
    q = q_ref[...]
    out_ref[...] = jnp.sum(q * q, axis=1)[None, :]


def _s0(queries):
    return pl.pallas_call(
        _s0_body,
        out_shape=jax.ShapeDtypeStruct((1, Q), jnp.float32),
    )(queries)


def _s1_body(qt_ref, qsq_ref, k_ref, mb_ref, ma_ref, ksq_ref):
    step = pl.program_id(0)
    qt = qt_ref[...]
    qsq = qsq_ref[...]
    k = k_ref[...]
    k_sq = jnp.sum(k * k, axis=1, keepdims=True)
    dots = lax.dot_general(k, qt, (((1,), (0,)), ((), ())),
                           preferred_element_type=jnp.float32)
    dist = (qsq - 2.0 * dots) + k_sq
    row = step * KT + lax.broadcasted_iota(jnp.int32, (KT, 1), 0)
    dist = jnp.where(row < K_TOTAL, dist, BIG)
    bmin = jnp.min(dist.reshape(KT // CB, CB, Q), axis=1)
    mb_ref[0] = jnp.swapaxes(bmin, 0, 1)
    ma_ref[0] = jnp.min(bmin.reshape(KT // CA, CA // CB, Q), axis=1)
    ksq_ref[0] = k_sq


def _s1(queries_t, q_sq, keys_p):
    return pl.pallas_call(
        _s1_body,
        grid=(NSTEP,),
        in_specs=[
            pl.BlockSpec((D, Q), lambda i: (0, 0)),
            pl.BlockSpec((1, Q), lambda i: (0, 0)),
            pl.BlockSpec((KT, D), lambda i: (i, 0)),
        ],
        out_specs=[
            pl.BlockSpec((1, Q, KT // CB), lambda i: (i, 0, 0)),
            pl.BlockSpec((1, KT // CA, Q), lambda i: (i, 0, 0)),
            pl.BlockSpec((1, KT, 1), lambda i: (i, 0, 0)),
        ],
        out_shape=[
            jax.ShapeDtypeStruct((NSTEP, Q, KT // CB), jnp.float32),
            jax.ShapeDtypeStruct((NSTEP, KT // CA, Q), jnp.float32),
            jax.ShapeDtypeStruct((NSTEP, KT, 1), jnp.float32),
        ],
    )(queries_t, q_sq, keys_p)


def _extract_body(v_ref, g_ref, outv_ref, outi_ref):
    V = v_ref[...]
    G = g_ref[...]
    for i in range(TOPK):
        m = jnp.min(V, axis=0)
        hit = V == m[None, :]
        gi = jnp.min(jnp.where(hit, G, IBIG), axis=0)
        outv_ref[i] = m
        outi_ref[i] = gi
        V = jnp.where(hit & (G == gi[None, :]), BIG, V)


def _extract(vals_t, ids_t):
    return pl.pallas_call(
        _extract_body,
        out_shape=[
            jax.ShapeDtypeStruct((TOPK, Q), jnp.float32),
            jax.ShapeDtypeStruct((TOPK, Q), jnp.int32),
        ],
    )(vals_t, ids_t)


def _sc_gather(table, idx, width, chunk):
    b = idx.shape[0]
    info = plsc.get_sparse_core_info()
    nw = info.num_cores * info.num_subcores
    n = b // nw
    mesh = plsc.VectorSubcoreMesh(core_axis_name="c", subcore_axis_name="s")

    @functools.partial(
        pl.kernel, mesh=mesh,
        compiler_params=pltpu.CompilerParams(use_tc_tiling_on_sc=False),
        out_type=jax.ShapeDtypeStruct((b, width), jnp.float32),
        scratch_types=[
            pltpu.VMEM((chunk,), jnp.int32),
            pltpu.VMEM((chunk, width), jnp.float32),
            pltpu.SemaphoreType.DMA,
        ],
    )
    def k(idx_hbm, table_hbm, out_hbm, idx_v, rows_v, sem):
        wid = lax.axis_index("s") * info.num_cores + lax.axis_index("c")
        base = wid * n
        for j in range(n // chunk):
            off = base + j * chunk
            pltpu.sync_copy(idx_hbm.at[pl.ds(off, chunk)], idx_v)
            pltpu.async_copy(table_hbm.at[idx_v], rows_v, sem).wait()
            pltpu.sync_copy(rows_v, out_hbm.at[pl.ds(off, chunk)])

    return k(idx, table)


QB = 8
CBLK = QB * NCAND


def _s6a_body(c_ref, q_ref, ksq_ref, out_ref):
    cand = c_ref[...]
    qb = q_ref[...]
    ksq = ksq_ref[...]
    q_sq = jnp.sum(qb * qb, axis=1, keepdims=True)
    cand_t = jnp.swapaxes(cand, 0, 1)
    dots = lax.dot_general(qb, cand_t, (((1,), (0,)), ((), ())),
                           preferred_element_type=jnp.float32)
    t3 = dots.reshape(QB, QB, NCAND)
    ii = lax.broadcasted_iota(jnp.int32, (QB, QB, NCAND), 0)
    jj = lax.broadcasted_iota(jnp.int32, (QB, QB, NCAND), 1)
    own = jnp.sum(jnp.where(ii == jj, t3, 0.0), axis=1)
    out_ref[...] = (q_sq - 2.0 * own) + ksq


def _s6a(cand, queries, ksq_cand):
    return pl.pallas_call(
        _s6a_body,
        grid=(Q // QB,),
        in_specs=[
            pl.BlockSpec((CBLK, D), lambda i: (i, 0)),
            pl.BlockSpec((QB, D), lambda i: (i, 0)),
            pl.BlockSpec((QB, NCAND), lambda i: (i, 0)),
        ],
        out_specs=pl.BlockSpec((QB, NCAND), lambda i: (i, 0)),
        out_shape=jax.ShapeDtypeStruct((Q, NCAND), jnp.float32),
    )(cand, queries, ksq_cand)


def kernel(queries, keys, k):
    keys_p = jnp.pad(keys, ((0, K_PAD - K_TOTAL), (0, 0)))
    queries_t = queries.T

    q_sq = _s0(queries)

    m_b3, m_a3, ksq3 = _s1(queries_t, q_sq, keys_p)
    m_at = m_a3.reshape(NA, Q)

    a_ids = lax.broadcasted_iota(jnp.int32, (NA, Q), 0)
    _, sel_a_t = _extract(m_at, a_ids)
    sel_a = sel_a_t.T

    qcol = jnp.arange(Q, dtype=jnp.int32)[:, None]
    gb = (sel_a[:, :, None] * CB +
          jnp.arange(CB, dtype=jnp.int32)[None, None, :]).reshape(Q, NCAND)
    bm_idx = ((sel_a // APS) * (Q * APS) + qcol * APS + sel_a % APS).reshape(-1)
    bm = _sc_gather(m_b3.reshape(NSTEP * Q * APS, CB), bm_idx, CB, 1024)
    bm_t = bm.reshape(Q, NCAND).T

    _, sel_b_t = _extract(bm_t, gb.T)
    sel_b = sel_b_t.T

    ck = (sel_b[:, :, None] * CB +
          jnp.arange(CB, dtype=jnp.int32)[None, None, :]).reshape(Q, NCAND)
    ck_flat = ck.reshape(-1)
    cand = _sc_gather(keys_p, ck_flat, D, 2048)
    ksq_cand = _sc_gather(ksq3.reshape(NB, CB), sel_b.reshape(-1), CB, 1024)

    dist_c = _s6a(cand, queries, ksq_cand.reshape(Q, NCAND))
    vals_t, idx_t = _extract(dist_c.T, ck.T)
    top_vals = vals_t.T
    top_idx = idx_t.T + (k * 0)

    gathered = _sc_gather(keys_p, top_idx.reshape(-1), D, 1024)
    return gathered.reshape(Q, TOPK, D), top_vals, top_idx
